# baseline (speedup 1.0000x reference)
import jax
import jax.numpy as jnp
from jax.experimental import pallas as pl
from jax.experimental.pallas import tpu as pltpu

B = 32
ZS = 128
CC = 16
N_OUT = 65536
N_CHUNKS = 1
CHUNK = N_OUT // N_CHUNKS


def _leaky(x):
    return jnp.where(x > 0, x, 0.2 * x)


def _dot_t(a, b_t):
    return jax.lax.dot_general(a, b_t, (((1,), (1,)), ((), ())),
                               preferred_element_type=jnp.float32)


def _fgl_kernel(s_hbm, t_hbm, c_hbm,
                fc0b_hbm, fc1b_hbm, fc2b_hbm, fc3b_hbm, fc4b_hbm,
                up0b_hbm, up1b_hbm, up2b_hbm, up3b_hbm, up4b_hbm,
                up4w_hbm, z_hbm, Es_hbm, Et_hbm, Ec_hbm,
                fc0_hbm, fc1_hbm, fc2_hbm, fc3_hbm, fc4_hbm,
                up0_hbm, up1_hbm, up2_hbm, up3_hbm,
                out_ref,
                s_v, t_v, c_v,
                fc0b_v, fc1b_v, fc2b_v, fc3b_v, fc4b_v,
                up0b_v, up1b_v, up2b_v, up3b_v, up4b_v,
                up4w_v, z_v, Es_v, Et_v, Ec_v,
                fc0_v, fc1_v, fc2_v, fc3_v, fc4_v,
                up0_v, up1_v, up2_v, up3_v,
                y_ref, sem):
    f32 = jnp.float32

    @pl.when(pl.program_id(0) == 0)
    def _compute_chain():
        hbm_refs = [s_hbm, t_hbm, c_hbm,
                    fc0b_hbm, fc1b_hbm, fc2b_hbm, fc3b_hbm, fc4b_hbm,
                    up0b_hbm, up1b_hbm, up2b_hbm, up3b_hbm, up4b_hbm,
                    up4w_hbm, z_hbm, Es_hbm, Et_hbm, Ec_hbm,
                    fc0_hbm, fc1_hbm, fc2_hbm, fc3_hbm, fc4_hbm,
                    up0_hbm, up1_hbm, up2_hbm, up3_hbm]
        vmem_refs = [s_v, t_v, c_v,
                     fc0b_v, fc1b_v, fc2b_v, fc3b_v, fc4b_v,
                     up0b_v, up1b_v, up2b_v, up3b_v, up4b_v,
                     up4w_v, z_v, Es_v, Et_v, Ec_v,
                     fc0_v, fc1_v, fc2_v, fc3_v, fc4_v,
                     up0_v, up1_v, up2_v, up3_v]
        copies = [pltpu.make_async_copy(s, d, sem.at[i])
                  for i, (s, d) in enumerate(zip(hbm_refs, vmem_refs))]
        for c in copies:
            c.start()
        for c in copies:
            c.wait()

        def idx_col(ref):
            return jnp.transpose(ref[:].reshape(1, B))

        def onehot(col, n):
            iota = jax.lax.broadcasted_iota(jnp.int32, (B, n), 1)
            return (iota == col).astype(f32)

        def rowb(ref, w):
            return jnp.broadcast_to(ref[:][None, :], (B, w))

        se = _dot_t(onehot(idx_col(s_v), 64), Es_v[:, :])
        te = _dot_t(onehot(idx_col(t_v), 128), Et_v[:, :])
        ce = _dot_t(onehot(idx_col(c_v), 256), Ec_v[:, :])
        cat3 = jnp.concatenate([se, te, ce], axis=1)

        c0 = se @ fc0_v[:, :] + rowb(fc0b_v, CC)
        c1 = _dot_t(jnp.concatenate([se, te], axis=1), fc1_v[:, :]) + rowb(fc1b_v, CC)
        c2 = _dot_t(cat3, fc2_v[:, :]) + rowb(fc2b_v, CC)
        c3 = _dot_t(cat3, fc3_v[:, :]) + rowb(fc3b_v, CC)
        c4 = _dot_t(cat3, fc4_v[:, :]) + rowb(fc4b_v, CC)

        x = z_v[:, :]
        x = _leaky(_dot_t(jnp.concatenate([x, c0], axis=1), up0_v[:, :]) + rowb(up0b_v, 64))
        x = _leaky(_dot_t(jnp.concatenate([x, c1], axis=1), up1_v[:, :]) + rowb(up1b_v, 32))
        x = _leaky(_dot_t(jnp.concatenate([x, c2], axis=1), up2_v[:, :]) + rowb(up2b_v, 16))
        x = _leaky(_dot_t(jnp.concatenate([x, c3], axis=1), up3_v[:, :]) + rowb(up3b_v, 8))
        x4 = jnp.concatenate([x, c4], axis=1)
        w4 = jnp.broadcast_to(up4w_v[:][None, :], (B, 24))
        y = jnp.sum(x4 * w4, axis=1, keepdims=True) + rowb(up4b_v, 1)
        y_ref[:, :] = y

    yv = y_ref[:, :]
    out_ref[:, :, :] = jnp.broadcast_to(yv[:, :, None], (B, CHUNK // 128, 128))


def kernel(z, studies, tasks, contrasts, Es, Et, Ec,
           fc0_W, fc0_b, fc1_W, fc1_b, fc2_W, fc2_b, fc3_W, fc3_b,
           fc4_W, fc4_b, up0_W, up0_b, up1_W, up1_b, up2_W, up2_b,
           up3_W, up3_b, up4_W, up4_b, idx0, idx1, idx2, idx3, idx4):
    args = (studies, tasks, contrasts,
            fc0_b, fc1_b, fc2_b, fc3_b, fc4_b,
            up0_b, up1_b, up2_b, up3_b, up4_b,
            up4_W.ravel(), z, Es.T, Et.T, Ec.T,
            fc0_W, fc1_W.T, fc2_W.T, fc3_W.T, fc4_W.T,
            up0_W.T, up1_W.T, up2_W.T, up3_W.T)
    args = tuple(a if a.size == 1
                 else pltpu.with_memory_space_constraint(a, pltpu.MemorySpace.HBM)
                 for a in args)
    hbm = pl.BlockSpec(memory_space=pltpu.MemorySpace.HBM)
    out = pl.pallas_call(
        _fgl_kernel,
        grid=(N_CHUNKS,),
        in_specs=[hbm] * len(args),
        out_specs=pl.BlockSpec((B, CHUNK // 128, 128), lambda i: (0, i, 0)),
        out_shape=jax.ShapeDtypeStruct((B, N_OUT // 128, 128), jnp.float32),
        scratch_shapes=[
            pltpu.VMEM((B,), jnp.int32),
            pltpu.VMEM((B,), jnp.int32),
            pltpu.VMEM((B,), jnp.int32),
            pltpu.VMEM((CC,), jnp.float32),
            pltpu.VMEM((CC,), jnp.float32),
            pltpu.VMEM((CC,), jnp.float32),
            pltpu.VMEM((CC,), jnp.float32),
            pltpu.VMEM((CC,), jnp.float32),
            pltpu.VMEM((64,), jnp.float32),
            pltpu.VMEM((32,), jnp.float32),
            pltpu.VMEM((16,), jnp.float32),
            pltpu.VMEM((8,), jnp.float32),
            pltpu.VMEM((1,), jnp.float32),
            pltpu.VMEM((24,), jnp.float32),
            pltpu.VMEM((B, ZS), jnp.float32),
            pltpu.VMEM((CC, 64), jnp.float32),
            pltpu.VMEM((CC, 128), jnp.float32),
            pltpu.VMEM((CC, 256), jnp.float32),
            pltpu.VMEM((CC, CC), jnp.float32),
            pltpu.VMEM((CC, 2 * CC), jnp.float32),
            pltpu.VMEM((CC, 3 * CC), jnp.float32),
            pltpu.VMEM((CC, 3 * CC), jnp.float32),
            pltpu.VMEM((CC, 3 * CC), jnp.float32),
            pltpu.VMEM((64, ZS + CC), jnp.float32),
            pltpu.VMEM((32, 64 + CC), jnp.float32),
            pltpu.VMEM((16, 32 + CC), jnp.float32),
            pltpu.VMEM((8, 16 + CC), jnp.float32),
            pltpu.VMEM((B, 1), jnp.float32),
            pltpu.SemaphoreType.DMA((27,)),
        ],
    )(*args)
    return out.reshape(B, N_OUT, 1)


# speedup vs baseline: 1.0195x; 1.0195x over previous
import jax
import jax.numpy as jnp
from jax.experimental import pallas as pl
from jax.experimental.pallas import tpu as pltpu

B = 32
ZS = 128
CC = 16
N_OUT = 65536
N_CHUNKS = 2
CHUNK = N_OUT // N_CHUNKS


def _leaky(x):
    return jnp.where(x > 0, x, 0.2 * x)


def _dot_t(a, b_t):
    return jax.lax.dot_general(a, b_t, (((1,), (1,)), ((), ())),
                               preferred_element_type=jnp.float32)


def _fgl_kernel(s_hbm, t_hbm, c_hbm,
                fc0b_hbm, fc1b_hbm, fc2b_hbm, fc3b_hbm, fc4b_hbm,
                up0b_hbm, up1b_hbm, up2b_hbm, up3b_hbm, up4b_hbm,
                up4w_hbm, z_hbm, Es_hbm, Et_hbm, Ec_hbm,
                fc0_hbm, fc1_hbm, fc2_hbm, fc3_hbm, fc4_hbm,
                up0_hbm, up1_hbm, up2_hbm, up3_hbm,
                out_ref,
                s_v, t_v, c_v,
                fc0b_v, fc1b_v, fc2b_v, fc3b_v, fc4b_v,
                up0b_v, up1b_v, up2b_v, up3b_v, up4b_v,
                up4w_v, z_v, Es_v, Et_v, Ec_v,
                fc0_v, fc1_v, fc2_v, fc3_v, fc4_v,
                up0_v, up1_v, up2_v, up3_v,
                y_ref, sem):
    f32 = jnp.float32

    @pl.when(pl.program_id(0) == 0)
    def _compute_chain():
        hbm_refs = [s_hbm, t_hbm, c_hbm,
                    fc0b_hbm, fc1b_hbm, fc2b_hbm, fc3b_hbm, fc4b_hbm,
                    up0b_hbm, up1b_hbm, up2b_hbm, up3b_hbm, up4b_hbm,
                    up4w_hbm, z_hbm, Es_hbm, Et_hbm, Ec_hbm,
                    fc0_hbm, fc1_hbm, fc2_hbm, fc3_hbm, fc4_hbm,
                    up0_hbm, up1_hbm, up2_hbm, up3_hbm]
        vmem_refs = [s_v, t_v, c_v,
                     fc0b_v, fc1b_v, fc2b_v, fc3b_v, fc4b_v,
                     up0b_v, up1b_v, up2b_v, up3b_v, up4b_v,
                     up4w_v, z_v, Es_v, Et_v, Ec_v,
                     fc0_v, fc1_v, fc2_v, fc3_v, fc4_v,
                     up0_v, up1_v, up2_v, up3_v]
        copies = [pltpu.make_async_copy(s, d, sem.at[i])
                  for i, (s, d) in enumerate(zip(hbm_refs, vmem_refs))]
        for c in copies:
            c.start()
        for c in copies:
            c.wait()

        def idx_col(ref):
            return jnp.transpose(ref[:].reshape(1, B))

        def onehot(col, n):
            iota = jax.lax.broadcasted_iota(jnp.int32, (B, n), 1)
            return (iota == col).astype(f32)

        def rowb(ref, w):
            return jnp.broadcast_to(ref[:][None, :], (B, w))

        se = _dot_t(onehot(idx_col(s_v), 64), Es_v[:, :])
        te = _dot_t(onehot(idx_col(t_v), 128), Et_v[:, :])
        ce = _dot_t(onehot(idx_col(c_v), 256), Ec_v[:, :])
        cat3 = jnp.concatenate([se, te, ce], axis=1)

        c0 = se @ fc0_v[:, :] + rowb(fc0b_v, CC)
        c1 = _dot_t(jnp.concatenate([se, te], axis=1), fc1_v[:, :]) + rowb(fc1b_v, CC)
        c2 = _dot_t(cat3, fc2_v[:, :]) + rowb(fc2b_v, CC)
        c3 = _dot_t(cat3, fc3_v[:, :]) + rowb(fc3b_v, CC)
        c4 = _dot_t(cat3, fc4_v[:, :]) + rowb(fc4b_v, CC)

        x = z_v[:, :]
        x = _leaky(_dot_t(jnp.concatenate([x, c0], axis=1), up0_v[:, :]) + rowb(up0b_v, 64))
        x = _leaky(_dot_t(jnp.concatenate([x, c1], axis=1), up1_v[:, :]) + rowb(up1b_v, 32))
        x = _leaky(_dot_t(jnp.concatenate([x, c2], axis=1), up2_v[:, :]) + rowb(up2b_v, 16))
        x = _leaky(_dot_t(jnp.concatenate([x, c3], axis=1), up3_v[:, :]) + rowb(up3b_v, 8))
        x4 = jnp.concatenate([x, c4], axis=1)
        w4 = jnp.broadcast_to(up4w_v[:][None, :], (B, 24))
        y = jnp.sum(x4 * w4, axis=1, keepdims=True) + rowb(up4b_v, 1)
        y_ref[:, :] = y

    yv = y_ref[:, :]
    out_ref[:, :, :] = jnp.broadcast_to(yv[:, :, None], (B, CHUNK // 128, 128))


def kernel(z, studies, tasks, contrasts, Es, Et, Ec,
           fc0_W, fc0_b, fc1_W, fc1_b, fc2_W, fc2_b, fc3_W, fc3_b,
           fc4_W, fc4_b, up0_W, up0_b, up1_W, up1_b, up2_W, up2_b,
           up3_W, up3_b, up4_W, up4_b, idx0, idx1, idx2, idx3, idx4):
    args = (studies, tasks, contrasts,
            fc0_b, fc1_b, fc2_b, fc3_b, fc4_b,
            up0_b, up1_b, up2_b, up3_b, up4_b,
            up4_W.ravel(), z, Es.T, Et.T, Ec.T,
            fc0_W, fc1_W.T, fc2_W.T, fc3_W.T, fc4_W.T,
            up0_W.T, up1_W.T, up2_W.T, up3_W.T)
    args = tuple(a if a.size == 1
                 else pltpu.with_memory_space_constraint(a, pltpu.MemorySpace.HBM)
                 for a in args)
    hbm = pl.BlockSpec(memory_space=pltpu.MemorySpace.HBM)
    out = pl.pallas_call(
        _fgl_kernel,
        grid=(N_CHUNKS,),
        in_specs=[hbm] * len(args),
        out_specs=pl.BlockSpec((B, CHUNK // 128, 128), lambda i: (0, i, 0)),
        out_shape=jax.ShapeDtypeStruct((B, N_OUT // 128, 128), jnp.float32),
        scratch_shapes=[
            pltpu.VMEM((B,), jnp.int32),
            pltpu.VMEM((B,), jnp.int32),
            pltpu.VMEM((B,), jnp.int32),
            pltpu.VMEM((CC,), jnp.float32),
            pltpu.VMEM((CC,), jnp.float32),
            pltpu.VMEM((CC,), jnp.float32),
            pltpu.VMEM((CC,), jnp.float32),
            pltpu.VMEM((CC,), jnp.float32),
            pltpu.VMEM((64,), jnp.float32),
            pltpu.VMEM((32,), jnp.float32),
            pltpu.VMEM((16,), jnp.float32),
            pltpu.VMEM((8,), jnp.float32),
            pltpu.VMEM((1,), jnp.float32),
            pltpu.VMEM((24,), jnp.float32),
            pltpu.VMEM((B, ZS), jnp.float32),
            pltpu.VMEM((CC, 64), jnp.float32),
            pltpu.VMEM((CC, 128), jnp.float32),
            pltpu.VMEM((CC, 256), jnp.float32),
            pltpu.VMEM((CC, CC), jnp.float32),
            pltpu.VMEM((CC, 2 * CC), jnp.float32),
            pltpu.VMEM((CC, 3 * CC), jnp.float32),
            pltpu.VMEM((CC, 3 * CC), jnp.float32),
            pltpu.VMEM((CC, 3 * CC), jnp.float32),
            pltpu.VMEM((64, ZS + CC), jnp.float32),
            pltpu.VMEM((32, 64 + CC), jnp.float32),
            pltpu.VMEM((16, 32 + CC), jnp.float32),
            pltpu.VMEM((8, 16 + CC), jnp.float32),
            pltpu.VMEM((B, 1), jnp.float32),
            pltpu.SemaphoreType.DMA((27,)),
        ],
    )(*args)
    return out.reshape(B, N_OUT, 1)
